# baseline (speedup 1.0000x reference)
import functools

import jax
import jax.numpy as jnp
from jax import lax
from jax.experimental import pallas as pl
from jax.experimental.pallas import tpu as pltpu
from jax.experimental.pallas import tpu_sc as plsc

_BIG = 1e30
_TOPK = 20
_L = 16
_BK = 2048
_C = 14336
_NCHUNK = 7
_SEED_CHUNKS = 1
_SEED_BLOCKS = _SEED_CHUNKS * (_C // _BK)
_G = 8
_NG1 = _C // (_G * _L)
_NG2 = _NG1 // _G
_CAPV = 128
_CAP = _CAPV * _L
_TW = 128


def _dist_body(q_ref, b_ref, out_ref, taus_ref, r_ref, qn_ref, *,
               nb, tail, bk):
    kb = pl.program_id(0)

    @pl.when(kb == 0)
    def _init():
        q = q_ref[...]
        qn_ref[...] = jnp.sum(q * q, axis=1, keepdims=True)
        r_ref[...] = jnp.full(r_ref.shape, _BIG, jnp.float32)

    b = b_ref[...]
    bn = jnp.sum(b * b, axis=1)[None, :]
    prod = lax.dot_general(q_ref[...], b, (((1,), (1,)), ((), ())),
                           preferred_element_type=jnp.float32)
    d = jnp.maximum(qn_ref[...] + bn - 2.0 * prod, 0.0)
    col = lax.broadcasted_iota(jnp.int32, d.shape, 1)
    d = jnp.where((kb < nb - 1) | (col < tail), d, _BIG)
    out_ref[...] = d

    @pl.when(kb < _SEED_BLOCKS)
    def _seed():
        r = r_ref[...]
        ms = []
        m = jnp.minimum(jnp.min(d, axis=1, keepdims=True),
                        jnp.min(r, axis=1, keepdims=True))
        ms.append(m)
        for _ in range(_TOPK - 1):
            md = jnp.min(jnp.where(d > m, d, _BIG), axis=1, keepdims=True)
            mr = jnp.min(jnp.where(r > m, r, _BIG), axis=1, keepdims=True)
            m = jnp.minimum(md, mr)
            ms.append(m)
        tops = jnp.concatenate(ms, axis=1)
        r_ref[:, :_TOPK] = tops

    @pl.when(kb == _SEED_BLOCKS - 1)
    def _emit_taus():
        taus_ref[...] = r_ref[...]


def _scalar(v):
    return lax.squeeze(lax.slice(v, (0,), (1,)), dimensions=(0,))


_GATHER_DN = lax.GatherDimensionNumbers(
    offset_dims=(), collapsed_slice_dims=(0,), start_index_map=(0,))


def _lane(v, el):
    iota = lax.broadcasted_iota(jnp.int32, (_L,), 0)
    idx = jnp.bitwise_and(iota + el, _L - 1)
    s = lax.gather(v, idx[:, None], dimension_numbers=_GATHER_DN,
                   slice_sizes=(1,),
                   mode=lax.GatherScatterMode.PROMISE_IN_BOUNDS)
    return _scalar(s)


def _rd(ref, base, el):
    return _lane(ref[pl.ds(base, _L)], el)


def _sc_body(dist_ref, taus_ref, cand_ref, buf, minsbuf, mins2, m3buf,
             tbuf, cand, sem0, sem1, tsem, *, rows_per):
    nc = 2
    wid = lax.axis_index("s") * nc + lax.axis_index("c")
    sems = (sem0, sem1)
    bigs = jnp.full((_L,), _BIG, jnp.float32)
    row0 = wid * rows_per
    base = _SEED_CHUNKS * _C

    iota = lax.broadcasted_iota(jnp.int32, (_L,), 0)

    def row_body(r_local, _):
        row = row0 + r_local
        pltpu.sync_copy(taus_ref.at[row, pl.ds(0, 2 * _L)],
                        tbuf.at[pl.ds(0, 2 * _L)])
        tv = tbuf[pl.ds(_L, _L)]
        tau = _scalar(lax.slice(tv, (_TOPK - 1 - _L,), (_TOPK - _L,)))
        pltpu.async_copy(dist_ref.at[row, pl.ds(base, _C)],
                         buf.at[0, pl.ds(0, _C)], sems[0])
        def creset(i, _):
            cand[pl.ds(i * _L, _L)] = bigs
            return 0

        lax.fori_loop(0, _CAPV, creset, 0)
        cnt = jnp.int32(0)
        for cc in range(_NCHUNK - _SEED_CHUNKS):
            p = cc % 2
            if cc + 1 < _NCHUNK - _SEED_CHUNKS:
                pltpu.async_copy(
                    dist_ref.at[row, pl.ds(base + (cc + 1) * _C, _C)],
                    buf.at[(cc + 1) % 2, pl.ds(0, _C)], sems[(cc + 1) % 2])
            pltpu.make_async_copy(dist_ref.at[row, pl.ds(base + cc * _C, _C)],
                                  buf.at[p, pl.ds(0, _C)], sems[p]).wait()
            bufp = buf.at[p]

            def l1(g, _):
                vs = [bufp[pl.ds((g * _G + u) * _L, _L)] for u in range(_G)]
                m0 = jnp.minimum(jnp.minimum(vs[0], vs[1]),
                                 jnp.minimum(vs[2], vs[3]))
                m1 = jnp.minimum(jnp.minimum(vs[4], vs[5]),
                                 jnp.minimum(vs[6], vs[7]))
                minsbuf[pl.ds(g * _L, _L)] = jnp.minimum(m0, m1)
                return 0

            lax.fori_loop(0, _NG1, l1, 0)

            def l2(s, _):
                vs = [minsbuf[pl.ds((s * _G + u) * _L, _L)]
                      for u in range(_G)]
                m0 = jnp.minimum(jnp.minimum(vs[0], vs[1]),
                                 jnp.minimum(vs[2], vs[3]))
                m1 = jnp.minimum(jnp.minimum(vs[4], vs[5]),
                                 jnp.minimum(vs[6], vs[7]))
                mins2[pl.ds(s * _L, _L)] = jnp.minimum(m0, m1)
                return 0

            lax.fori_loop(0, _NG2, l2, 0)

            m3 = mins2[pl.ds(0, _L)]
            for s in range(1, _NG2):
                m3 = jnp.minimum(m3, mins2[pl.ds(s * _L, _L)])
            m3buf[pl.ds(0, _L)] = m3

            def lane_body(el, cnt):
                def drill_s(s, cnt):
                    def drill_g(u, cnt):
                        g = s * _G + u

                        def drill_v(w, cnt):
                            v = bufp[pl.ds((g * _G + w) * _L, _L)]
                            val = _lane(v, el)

                            def app(c):
                                slot = jnp.minimum(c, _CAPV - 1)
                                cand[pl.ds(slot * _L, _L)] = jnp.where(
                                    iota == 0, jnp.broadcast_to(val, (_L,)),
                                    bigs)
                                return c + 1

                            return lax.cond(val < tau, app, lambda c: c, cnt)

                        return lax.cond(_rd(minsbuf, g * _L, el) < tau,
                                        lambda c: lax.fori_loop(
                                            0, _G, drill_v, c),
                                        lambda c: c, cnt)

                    return lax.cond(_rd(mins2, s * _L, el) < tau,
                                    lambda c: lax.fori_loop(
                                        0, _G, drill_g, c),
                                    lambda c: c, cnt)

                return lax.cond(_rd(m3buf, 0, el) < tau,
                                lambda c: lax.fori_loop(0, _NG2, drill_s, c),
                                lambda c: c, cnt)

            cnt = lax.fori_loop(0, _L, lane_body, cnt)

        pltpu.async_copy(cand.at[pl.ds(0, _CAP)], cand_ref.at[row],
                         tsem).wait()
        return 0

    lax.fori_loop(0, rows_per, row_body, 0)


def _merge_body(cand_ref, taus_ref, scale_ref, out_ref):
    d = cand_ref[...]
    r = taus_ref[...]
    ms = []
    m = jnp.minimum(jnp.min(d, axis=1, keepdims=True),
                    jnp.min(r, axis=1, keepdims=True))
    ms.append(m)
    for _ in range(_TOPK - 1):
        md = jnp.min(jnp.where(d > m, d, _BIG), axis=1, keepdims=True)
        mr = jnp.min(jnp.where(r > m, r, _BIG), axis=1, keepdims=True)
        m = jnp.minimum(md, mr)
        ms.append(m)
    tops = jnp.concatenate(ms, axis=1)
    kern = jnp.exp(tops * -0.5)
    out_ref[...] = jnp.sum(kern * scale_ref[0:1, :_TOPK], axis=1,
                           keepdims=True)


@jax.jit
def kernel(real_features, buffer_features, k):
    n, dim = real_features.shape
    kbuf = buffer_features.shape[0]
    bk = _BK
    kpad = _NCHUNK * _C
    nb = kpad // bk
    tail = kbuf - (nb - 1) * bk

    dist_body = functools.partial(_dist_body, nb=nb, tail=tail, bk=bk)
    dists, taus = pl.pallas_call(
        dist_body,
        grid=(nb,),
        in_specs=[
            pl.BlockSpec((n, dim), lambda i: (0, 0)),
            pl.BlockSpec((bk, dim), lambda i: (i, 0)),
        ],
        out_specs=[
            pl.BlockSpec((n, bk), lambda i: (0, i)),
            pl.BlockSpec((n, _TW), lambda i: (0, 0)),
        ],
        out_shape=[
            jax.ShapeDtypeStruct((n, kpad), jnp.float32),
            jax.ShapeDtypeStruct((n, _TW), jnp.float32),
        ],
        scratch_shapes=[
            pltpu.VMEM((n, _TW), jnp.float32),
            pltpu.VMEM((n, 1), jnp.float32),
        ],
    )(real_features, buffer_features)

    nworkers = 32
    rows_per = n // nworkers
    mesh = plsc.VectorSubcoreMesh(core_axis_name="c", subcore_axis_name="s")
    sc_body = functools.partial(_sc_body, rows_per=rows_per)
    cand = pl.kernel(
        sc_body,
        out_type=jax.ShapeDtypeStruct((n, _CAP), jnp.float32),
        mesh=mesh,
        scratch_types=[
            pltpu.VMEM((2, _C + _L), jnp.float32),
            pltpu.VMEM((_NG1 * _L + _L,), jnp.float32),
            pltpu.VMEM((_NG2 * _L + _L,), jnp.float32),
            pltpu.VMEM((2 * _L,), jnp.float32),
            pltpu.VMEM((3 * _L,), jnp.float32),
            pltpu.VMEM((_CAP + _L,), jnp.float32),
            pltpu.SemaphoreType.DMA,
            pltpu.SemaphoreType.DMA,
            pltpu.SemaphoreType.DMA,
        ],
    )(dists, taus)

    kf = jnp.asarray(k, jnp.float32)
    idx = jnp.arange(_TW)
    scale = (jnp.where((idx < k) & (idx < _TOPK), 1.0, 0.0)
             .astype(jnp.float32) / kf)[None, :]

    out = pl.pallas_call(
        _merge_body,
        in_specs=[
            pl.BlockSpec((n, _CAP), lambda: (0, 0)),
            pl.BlockSpec((n, _TW), lambda: (0, 0)),
            pl.BlockSpec((1, _TW), lambda: (0, 0)),
        ],
        out_specs=pl.BlockSpec((n, 1), lambda: (0, 0)),
        out_shape=jax.ShapeDtypeStruct((n, 1), jnp.float32),
    )(cand, taus, scale)
    return out[:, 0]


# speedup vs baseline: 2.8380x; 2.8380x over previous
import functools

import jax
import jax.numpy as jnp
from jax import lax
from jax.experimental import pallas as pl
from jax.experimental.pallas import tpu as pltpu
from jax.experimental.pallas import tpu_sc as plsc

_BIG = 1e30
_TOPK = 20
_L = 16
_BK = 2048
_C = 14336
_NCHUNK = 7
_SEED_CHUNKS = 1
_SEED_BLOCKS = _SEED_CHUNKS * (_C // _BK)
_G = 8
_NG1 = _C // (_G * _L)
_NG2 = _NG1 // _G
_CAPV = 128
_CAP = _CAPV * _L
_TW = 128


def _dist_body(q_ref, b_ref, out_ref, taus_ref, r_ref, qn_ref, *,
               nb, tail, bk):
    kb = pl.program_id(0)

    @pl.when(kb == 0)
    def _init():
        q = q_ref[...]
        qn_ref[...] = jnp.sum(q * q, axis=1, keepdims=True)
        r_ref[...] = jnp.full(r_ref.shape, _BIG, jnp.float32)

    b = b_ref[...]
    bn = jnp.sum(b * b, axis=1)[None, :]
    prod = lax.dot_general(q_ref[...], b, (((1,), (1,)), ((), ())),
                           preferred_element_type=jnp.float32)
    d = jnp.maximum(qn_ref[...] + bn - 2.0 * prod, 0.0)
    col = lax.broadcasted_iota(jnp.int32, d.shape, 1)
    d = jnp.where((kb < nb - 1) | (col < tail), d, _BIG)
    out_ref[...] = d

    @pl.when(kb < _SEED_BLOCKS)
    def _seed():
        r = r_ref[...]
        ms = []
        m = jnp.minimum(jnp.min(d, axis=1, keepdims=True),
                        jnp.min(r, axis=1, keepdims=True))
        ms.append(m)
        for _ in range(_TOPK - 1):
            md = jnp.min(jnp.where(d > m, d, _BIG), axis=1, keepdims=True)
            mr = jnp.min(jnp.where(r > m, r, _BIG), axis=1, keepdims=True)
            m = jnp.minimum(md, mr)
            ms.append(m)
        tops = jnp.concatenate(ms, axis=1)
        r_ref[:, :_TOPK] = tops

    @pl.when(kb == _SEED_BLOCKS - 1)
    def _emit_taus():
        taus_ref[...] = r_ref[...]


def _scalar(v):
    return lax.squeeze(lax.slice(v, (0,), (1,)), dimensions=(0,))


_GATHER_DN = lax.GatherDimensionNumbers(
    offset_dims=(), collapsed_slice_dims=(0,), start_index_map=(0,))


def _lane(v, el):
    iota = lax.broadcasted_iota(jnp.int32, (_L,), 0)
    idx = jnp.bitwise_and(iota + el, _L - 1)
    s = lax.gather(v, idx[:, None], dimension_numbers=_GATHER_DN,
                   slice_sizes=(1,),
                   mode=lax.GatherScatterMode.PROMISE_IN_BOUNDS)
    return _scalar(s)


def _rd(ref, base, el):
    return _lane(ref[pl.ds(base, _L)], el)


def _sc_body(dist_ref, taus_ref, cand_ref, buf, minsbuf, mins2, m3buf,
             tbuf, cand, sem0, sem1, tsem, *, rows_per):
    nc = 2
    wid = lax.axis_index("s") * nc + lax.axis_index("c")
    sems = (sem0, sem1)
    bigs = jnp.full((_L,), _BIG, jnp.float32)
    row0 = wid * rows_per
    base = _SEED_CHUNKS * _C

    iota = lax.broadcasted_iota(jnp.int32, (_L,), 0)

    def row_body(r_local, _):
        row = row0 + r_local
        pltpu.sync_copy(taus_ref.at[row, pl.ds(0, 2 * _L)],
                        tbuf.at[pl.ds(0, 2 * _L)])
        tv = tbuf[pl.ds(_L, _L)]
        tau = _scalar(lax.slice(tv, (_TOPK - 1 - _L,), (_TOPK - _L,)))
        pltpu.async_copy(dist_ref.at[row, pl.ds(base, _C)],
                         buf.at[0, pl.ds(0, _C)], sems[0])
        def creset(i, _):
            cand[pl.ds(i * _L, _L)] = bigs
            return 0

        lax.fori_loop(0, _CAPV, creset, 0)
        cnt = jnp.int32(0)
        for cc in range(_NCHUNK - _SEED_CHUNKS):
            p = cc % 2
            if cc + 1 < _NCHUNK - _SEED_CHUNKS:
                pltpu.async_copy(
                    dist_ref.at[row, pl.ds(base + (cc + 1) * _C, _C)],
                    buf.at[(cc + 1) % 2, pl.ds(0, _C)], sems[(cc + 1) % 2])
            pltpu.make_async_copy(dist_ref.at[row, pl.ds(base + cc * _C, _C)],
                                  buf.at[p, pl.ds(0, _C)], sems[p]).wait()
            bufp = buf.at[p]

            def l1(g, _):
                vs = [bufp[pl.ds((g * _G + u) * _L, _L)] for u in range(_G)]
                m0 = jnp.minimum(jnp.minimum(vs[0], vs[1]),
                                 jnp.minimum(vs[2], vs[3]))
                m1 = jnp.minimum(jnp.minimum(vs[4], vs[5]),
                                 jnp.minimum(vs[6], vs[7]))
                minsbuf[pl.ds(g * _L, _L)] = jnp.minimum(m0, m1)
                return 0

            lax.fori_loop(0, _NG1, l1, 0)

            def l2(s, _):
                vs = [minsbuf[pl.ds((s * _G + u) * _L, _L)]
                      for u in range(_G)]
                m0 = jnp.minimum(jnp.minimum(vs[0], vs[1]),
                                 jnp.minimum(vs[2], vs[3]))
                m1 = jnp.minimum(jnp.minimum(vs[4], vs[5]),
                                 jnp.minimum(vs[6], vs[7]))
                mins2[pl.ds(s * _L, _L)] = jnp.minimum(m0, m1)
                return 0

            lax.fori_loop(0, _NG2, l2, 0)

            m3 = mins2[pl.ds(0, _L)]
            for s in range(1, _NG2):
                m3 = jnp.minimum(m3, mins2[pl.ds(s * _L, _L)])
            m3buf[pl.ds(0, _L)] = m3

            def lane_body(el, cnt):
                def drill_s(s, cnt):
                    def drill_g(u, cnt):
                        g = s * _G + u

                        def drill_v(w, cnt):
                            v = bufp[pl.ds((g * _G + w) * _L, _L)]
                            val = _lane(v, el)

                            def app(c):
                                slot = jnp.minimum(c, _CAPV - 1)
                                cand[pl.ds(slot * _L, _L)] = jnp.where(
                                    iota == 0, jnp.broadcast_to(val, (_L,)),
                                    bigs)
                                return c + 1

                            return lax.cond(val < tau, app, lambda c: c, cnt)

                        return lax.cond(_rd(minsbuf, g * _L, el) < tau,
                                        lambda c: lax.fori_loop(
                                            0, _G, drill_v, c),
                                        lambda c: c, cnt)

                    return lax.cond(_rd(mins2, s * _L, el) < tau,
                                    lambda c: lax.fori_loop(
                                        0, _G, drill_g, c),
                                    lambda c: c, cnt)

                return lax.cond(_rd(m3buf, 0, el) < tau,
                                lambda c: lax.fori_loop(0, _NG2, drill_s, c),
                                lambda c: c, cnt)

            cnt = lax.fori_loop(0, _L, lane_body, cnt)

        pltpu.async_copy(cand.at[pl.ds(0, _CAP)], cand_ref.at[row],
                         tsem).wait()
        return 0

    lax.fori_loop(0, rows_per, row_body, 0)


def _merge_body(cand_ref, taus_ref, scale_ref, out_ref):
    d = cand_ref[...]
    r = taus_ref[...]
    ms = []
    m = jnp.minimum(jnp.min(d, axis=1, keepdims=True),
                    jnp.min(r, axis=1, keepdims=True))
    ms.append(m)
    for _ in range(_TOPK - 1):
        md = jnp.min(jnp.where(d > m, d, _BIG), axis=1, keepdims=True)
        mr = jnp.min(jnp.where(r > m, r, _BIG), axis=1, keepdims=True)
        m = jnp.minimum(md, mr)
        ms.append(m)
    tops = jnp.concatenate(ms, axis=1)
    kern = jnp.exp(tops * -0.5)
    out_ref[...] = jnp.sum(kern * scale_ref[0:1, :_TOPK], axis=1,
                           keepdims=True)


def _fused_body(q_ref, b_ref, scale_ref, out_ref, r_ref, qn_ref, *,
                nb, tail, bk):
    kb = pl.program_id(0)

    @pl.when(kb == 0)
    def _init():
        q = q_ref[...]
        qn_ref[...] = jnp.sum(q * q, axis=1, keepdims=True)
        r_ref[...] = jnp.full(r_ref.shape, _BIG, jnp.float32)

    b = b_ref[...]
    bn = jnp.sum(b * b, axis=1)[None, :]
    prod = lax.dot_general(q_ref[...], b, (((1,), (1,)), ((), ())),
                           preferred_element_type=jnp.float32)
    d = jnp.maximum(qn_ref[...] + bn - 2.0 * prod, 0.0)
    col = lax.broadcasted_iota(jnp.int32, d.shape, 1)
    d = jnp.where((kb < nb - 1) | (col < tail), d, _BIG)
    r = r_ref[...]
    ms = []
    m = jnp.minimum(jnp.min(d, axis=1, keepdims=True),
                    jnp.min(r, axis=1, keepdims=True))
    ms.append(m)
    for _ in range(_TOPK - 1):
        md = jnp.min(jnp.where(d > m, d, _BIG), axis=1, keepdims=True)
        mr = jnp.min(jnp.where(r > m, r, _BIG), axis=1, keepdims=True)
        m = jnp.minimum(md, mr)
        ms.append(m)
    tops = jnp.concatenate(ms, axis=1)

    @pl.when(kb < nb - 1)
    def _carry():
        r_ref[:, :_TOPK] = tops

    @pl.when(kb == nb - 1)
    def _emit():
        kern = jnp.exp(tops * -0.5)
        out_ref[...] = jnp.sum(kern * scale_ref[0:1, :_TOPK], axis=1,
                               keepdims=True)


@jax.jit
def kernel(real_features, buffer_features, k):
    n, dim = real_features.shape
    kbuf = buffer_features.shape[0]
    bk = _BK
    kpad = _NCHUNK * _C
    nb = kpad // bk
    tail = kbuf - (nb - 1) * bk

    kf = jnp.asarray(k, jnp.float32)
    idx = jnp.arange(_TW)
    scale = (jnp.where((idx < k) & (idx < _TOPK), 1.0, 0.0)
             .astype(jnp.float32) / kf)[None, :]

    nsc = 320
    ntc = n - nsc
    q_sc = real_features[:nsc]
    q_tc = real_features[nsc:]

    dist_body = functools.partial(_dist_body, nb=nb, tail=tail, bk=bk)
    dists, taus = pl.pallas_call(
        dist_body,
        grid=(nb,),
        in_specs=[
            pl.BlockSpec((nsc, dim), lambda i: (0, 0)),
            pl.BlockSpec((bk, dim), lambda i: (i, 0)),
        ],
        out_specs=[
            pl.BlockSpec((nsc, bk), lambda i: (0, i)),
            pl.BlockSpec((nsc, _TW), lambda i: (0, 0)),
        ],
        out_shape=[
            jax.ShapeDtypeStruct((nsc, kpad), jnp.float32),
            jax.ShapeDtypeStruct((nsc, _TW), jnp.float32),
        ],
        scratch_shapes=[
            pltpu.VMEM((nsc, _TW), jnp.float32),
            pltpu.VMEM((nsc, 1), jnp.float32),
        ],
    )(q_sc, buffer_features)

    nworkers = 32
    rows_per = nsc // nworkers
    mesh = plsc.VectorSubcoreMesh(core_axis_name="c", subcore_axis_name="s")
    sc_body = functools.partial(_sc_body, rows_per=rows_per)
    cand = pl.kernel(
        sc_body,
        out_type=jax.ShapeDtypeStruct((nsc, _CAP), jnp.float32),
        mesh=mesh,
        scratch_types=[
            pltpu.VMEM((2, _C + _L), jnp.float32),
            pltpu.VMEM((_NG1 * _L + _L,), jnp.float32),
            pltpu.VMEM((_NG2 * _L + _L,), jnp.float32),
            pltpu.VMEM((2 * _L,), jnp.float32),
            pltpu.VMEM((3 * _L,), jnp.float32),
            pltpu.VMEM((_CAP + _L,), jnp.float32),
            pltpu.SemaphoreType.DMA,
            pltpu.SemaphoreType.DMA,
            pltpu.SemaphoreType.DMA,
        ],
    )(dists, taus)

    fused_body = functools.partial(_fused_body, nb=nb, tail=tail, bk=bk)
    out_tc = pl.pallas_call(
        fused_body,
        grid=(nb,),
        in_specs=[
            pl.BlockSpec((ntc, dim), lambda i: (0, 0)),
            pl.BlockSpec((bk, dim), lambda i: (i, 0)),
            pl.BlockSpec((1, _TW), lambda i: (0, 0)),
        ],
        out_specs=pl.BlockSpec((ntc, 1), lambda i: (0, 0)),
        out_shape=jax.ShapeDtypeStruct((ntc, 1), jnp.float32),
        scratch_shapes=[
            pltpu.VMEM((ntc, _TW), jnp.float32),
            pltpu.VMEM((ntc, 1), jnp.float32),
        ],
    )(q_tc, buffer_features, scale)

    out_sc = pl.pallas_call(
        _merge_body,
        in_specs=[
            pl.BlockSpec((nsc, _CAP), lambda: (0, 0)),
            pl.BlockSpec((nsc, _TW), lambda: (0, 0)),
            pl.BlockSpec((1, _TW), lambda: (0, 0)),
        ],
        out_specs=pl.BlockSpec((nsc, 1), lambda: (0, 0)),
        out_shape=jax.ShapeDtypeStruct((nsc, 1), jnp.float32),
    )(cand, taus, scale)
    return jnp.concatenate([out_sc[:, 0], out_tc[:, 0]])


# speedup vs baseline: 3.1621x; 1.1142x over previous
import functools

import jax
import jax.numpy as jnp
from jax import lax
from jax.experimental import pallas as pl
from jax.experimental.pallas import tpu as pltpu
from jax.experimental.pallas import tpu_sc as plsc

_BIG = 1e30
_TOPK = 20
_L = 16
_BK = 2048
_C = 14336
_NCHUNK = 7
_SEED_CHUNKS = 1
_SEED_BLOCKS = _SEED_CHUNKS * (_C // _BK)
_G = 8
_NG1 = _C // (_G * _L)
_NG2 = _NG1 // _G
_CAPV = 128
_CAP = _CAPV * _L
_TW = 128


def _dist_body(q_ref, b_ref, out_ref, taus_ref, r_ref, qn_ref, *,
               nb, tail, bk):
    kb = pl.program_id(0)

    @pl.when(kb == 0)
    def _init():
        q = q_ref[...]
        qn_ref[...] = jnp.sum(q * q, axis=1, keepdims=True)
        r_ref[...] = jnp.full(r_ref.shape, _BIG, jnp.float32)

    b = b_ref[...]
    bn = jnp.sum(b * b, axis=1)[None, :]
    prod = lax.dot_general(q_ref[...], b, (((1,), (1,)), ((), ())),
                           preferred_element_type=jnp.float32)
    d = jnp.maximum(qn_ref[...] + bn - 2.0 * prod, 0.0)
    col = lax.broadcasted_iota(jnp.int32, d.shape, 1)
    d = jnp.where((kb < nb - 1) | (col < tail), d, _BIG)
    out_ref[...] = d

    @pl.when(kb < _SEED_BLOCKS)
    def _seed():
        r = r_ref[...]
        ms = []
        m = jnp.minimum(jnp.min(d, axis=1, keepdims=True),
                        jnp.min(r, axis=1, keepdims=True))
        ms.append(m)
        for _ in range(_TOPK - 1):
            md = jnp.min(jnp.where(d > m, d, _BIG), axis=1, keepdims=True)
            mr = jnp.min(jnp.where(r > m, r, _BIG), axis=1, keepdims=True)
            m = jnp.minimum(md, mr)
            ms.append(m)
        tops = jnp.concatenate(ms, axis=1)
        r_ref[:, :_TOPK] = tops

    @pl.when(kb == _SEED_BLOCKS - 1)
    def _emit_taus():
        taus_ref[...] = r_ref[...]


def _scalar(v):
    return lax.squeeze(lax.slice(v, (0,), (1,)), dimensions=(0,))


_GATHER_DN = lax.GatherDimensionNumbers(
    offset_dims=(), collapsed_slice_dims=(0,), start_index_map=(0,))


def _lane(v, el):
    iota = lax.broadcasted_iota(jnp.int32, (_L,), 0)
    idx = jnp.bitwise_and(iota + el, _L - 1)
    s = lax.gather(v, idx[:, None], dimension_numbers=_GATHER_DN,
                   slice_sizes=(1,),
                   mode=lax.GatherScatterMode.PROMISE_IN_BOUNDS)
    return _scalar(s)


def _rd(ref, base, el):
    return _lane(ref[pl.ds(base, _L)], el)


def _sc_body(dist_ref, taus_ref, cand_ref, buf, minsbuf, mins2, m3buf,
             tbuf, cand, sem0, sem1, tsem, *, rows_per):
    nc = 2
    wid = lax.axis_index("s") * nc + lax.axis_index("c")
    sems = (sem0, sem1)
    bigs = jnp.full((_L,), _BIG, jnp.float32)
    row0 = wid * rows_per
    base = _SEED_CHUNKS * _C

    iota = lax.broadcasted_iota(jnp.int32, (_L,), 0)

    def row_body(r_local, _):
        row = row0 + r_local
        pltpu.sync_copy(taus_ref.at[row, pl.ds(0, 2 * _L)],
                        tbuf.at[pl.ds(0, 2 * _L)])
        tv = tbuf[pl.ds(_L, _L)]
        tau = _scalar(lax.slice(tv, (_TOPK - 1 - _L,), (_TOPK - _L,)))
        pltpu.async_copy(dist_ref.at[row, pl.ds(base, _C)],
                         buf.at[0, pl.ds(0, _C)], sems[0])
        def creset(i, _):
            cand[pl.ds(i * _L, _L)] = bigs
            return 0

        lax.fori_loop(0, _CAPV, creset, 0)
        cnt = jnp.int32(0)
        for cc in range(_NCHUNK - _SEED_CHUNKS):
            p = cc % 2
            if cc + 1 < _NCHUNK - _SEED_CHUNKS:
                pltpu.async_copy(
                    dist_ref.at[row, pl.ds(base + (cc + 1) * _C, _C)],
                    buf.at[(cc + 1) % 2, pl.ds(0, _C)], sems[(cc + 1) % 2])
            pltpu.make_async_copy(dist_ref.at[row, pl.ds(base + cc * _C, _C)],
                                  buf.at[p, pl.ds(0, _C)], sems[p]).wait()
            bufp = buf.at[p]

            def l1(g, _):
                vs = [bufp[pl.ds((g * _G + u) * _L, _L)] for u in range(_G)]
                m0 = jnp.minimum(jnp.minimum(vs[0], vs[1]),
                                 jnp.minimum(vs[2], vs[3]))
                m1 = jnp.minimum(jnp.minimum(vs[4], vs[5]),
                                 jnp.minimum(vs[6], vs[7]))
                minsbuf[pl.ds(g * _L, _L)] = jnp.minimum(m0, m1)
                return 0

            lax.fori_loop(0, _NG1, l1, 0)

            def l2(s, _):
                vs = [minsbuf[pl.ds((s * _G + u) * _L, _L)]
                      for u in range(_G)]
                m0 = jnp.minimum(jnp.minimum(vs[0], vs[1]),
                                 jnp.minimum(vs[2], vs[3]))
                m1 = jnp.minimum(jnp.minimum(vs[4], vs[5]),
                                 jnp.minimum(vs[6], vs[7]))
                mins2[pl.ds(s * _L, _L)] = jnp.minimum(m0, m1)
                return 0

            lax.fori_loop(0, _NG2, l2, 0)

            m3 = mins2[pl.ds(0, _L)]
            for s in range(1, _NG2):
                m3 = jnp.minimum(m3, mins2[pl.ds(s * _L, _L)])
            m3buf[pl.ds(0, _L)] = m3

            def lane_body(el, cnt):
                def drill_s(s, cnt):
                    def drill_g(u, cnt):
                        g = s * _G + u

                        def drill_v(w, cnt):
                            v = bufp[pl.ds((g * _G + w) * _L, _L)]
                            val = _lane(v, el)

                            def app(c):
                                slot = jnp.minimum(c, _CAPV - 1)
                                cand[pl.ds(slot * _L, _L)] = jnp.where(
                                    iota == 0, jnp.broadcast_to(val, (_L,)),
                                    bigs)
                                return c + 1

                            return lax.cond(val < tau, app, lambda c: c, cnt)

                        return lax.cond(_rd(minsbuf, g * _L, el) < tau,
                                        lambda c: lax.fori_loop(
                                            0, _G, drill_v, c),
                                        lambda c: c, cnt)

                    return lax.cond(_rd(mins2, s * _L, el) < tau,
                                    lambda c: lax.fori_loop(
                                        0, _G, drill_g, c),
                                    lambda c: c, cnt)

                return lax.cond(_rd(m3buf, 0, el) < tau,
                                lambda c: lax.fori_loop(0, _NG2, drill_s, c),
                                lambda c: c, cnt)

            cnt = lax.fori_loop(0, _L, lane_body, cnt)

        pltpu.async_copy(cand.at[pl.ds(0, _CAP)], cand_ref.at[row],
                         tsem).wait()
        return 0

    lax.fori_loop(0, rows_per, row_body, 0)


def _merge_body(cand_ref, taus_ref, scale_ref, out_ref):
    d = cand_ref[...]
    r = taus_ref[...]
    ms = []
    m = jnp.minimum(jnp.min(d, axis=1, keepdims=True),
                    jnp.min(r, axis=1, keepdims=True))
    ms.append(m)
    for _ in range(_TOPK - 1):
        md = jnp.min(jnp.where(d > m, d, _BIG), axis=1, keepdims=True)
        mr = jnp.min(jnp.where(r > m, r, _BIG), axis=1, keepdims=True)
        m = jnp.minimum(md, mr)
        ms.append(m)
    tops = jnp.concatenate(ms, axis=1)
    kern = jnp.exp(tops * -0.5)
    out_ref[...] = jnp.sum(kern * scale_ref[0:1, :_TOPK], axis=1,
                           keepdims=True)


def _fused_body(q_ref, b_ref, scale_ref, out_ref, r_ref, qn_ref, *,
                nb, tail, bk):
    kb = pl.program_id(0)

    @pl.when(kb == 0)
    def _init():
        q = q_ref[...]
        qn_ref[...] = jnp.sum(q * q, axis=1, keepdims=True)
        r_ref[...] = jnp.full(r_ref.shape, _BIG, jnp.float32)

    b = b_ref[...]
    bn = jnp.sum(b * b, axis=1)[None, :]
    prod = lax.dot_general(q_ref[...], b, (((1,), (1,)), ((), ())),
                           preferred_element_type=jnp.float32)
    d = jnp.maximum(qn_ref[...] + bn - 2.0 * prod, 0.0)
    col = lax.broadcasted_iota(jnp.int32, d.shape, 1)
    d = jnp.where((kb < nb - 1) | (col < tail), d, _BIG)
    r = r_ref[...]
    ms = []
    m = jnp.minimum(jnp.min(d, axis=1, keepdims=True),
                    jnp.min(r, axis=1, keepdims=True))
    ms.append(m)
    for _ in range(_TOPK - 1):
        md = jnp.min(jnp.where(d > m, d, _BIG), axis=1, keepdims=True)
        mr = jnp.min(jnp.where(r > m, r, _BIG), axis=1, keepdims=True)
        m = jnp.minimum(md, mr)
        ms.append(m)
    tops = jnp.concatenate(ms, axis=1)

    @pl.when(kb < nb - 1)
    def _carry():
        r_ref[:, :_TOPK] = tops

    @pl.when(kb == nb - 1)
    def _emit():
        kern = jnp.exp(tops * -0.5)
        out_ref[...] = jnp.sum(kern * scale_ref[0:1, :_TOPK], axis=1,
                               keepdims=True)


@jax.jit
def kernel(real_features, buffer_features, k):
    n, dim = real_features.shape
    kbuf = buffer_features.shape[0]
    bk = _BK
    kpad = _NCHUNK * _C
    nb = kpad // bk
    tail = kbuf - (nb - 1) * bk

    kf = jnp.asarray(k, jnp.float32)
    idx = jnp.arange(_TW)
    scale = (jnp.where((idx < k) & (idx < _TOPK), 1.0, 0.0)
             .astype(jnp.float32) / kf)[None, :]

    nsc = 288
    ntc = n - nsc
    q_sc = real_features[:nsc]
    q_tc = real_features[nsc:]

    dist_body = functools.partial(_dist_body, nb=nb, tail=tail, bk=bk)
    dists, taus = pl.pallas_call(
        dist_body,
        grid=(nb,),
        in_specs=[
            pl.BlockSpec((nsc, dim), lambda i: (0, 0)),
            pl.BlockSpec((bk, dim), lambda i: (i, 0)),
        ],
        out_specs=[
            pl.BlockSpec((nsc, bk), lambda i: (0, i)),
            pl.BlockSpec((nsc, _TW), lambda i: (0, 0)),
        ],
        out_shape=[
            jax.ShapeDtypeStruct((nsc, kpad), jnp.float32),
            jax.ShapeDtypeStruct((nsc, _TW), jnp.float32),
        ],
        scratch_shapes=[
            pltpu.VMEM((nsc, _TW), jnp.float32),
            pltpu.VMEM((nsc, 1), jnp.float32),
        ],
    )(q_sc, buffer_features)

    nworkers = 32
    rows_per = nsc // nworkers
    mesh = plsc.VectorSubcoreMesh(core_axis_name="c", subcore_axis_name="s")
    sc_body = functools.partial(_sc_body, rows_per=rows_per)
    cand = pl.kernel(
        sc_body,
        out_type=jax.ShapeDtypeStruct((nsc, _CAP), jnp.float32),
        mesh=mesh,
        scratch_types=[
            pltpu.VMEM((2, _C + _L), jnp.float32),
            pltpu.VMEM((_NG1 * _L + _L,), jnp.float32),
            pltpu.VMEM((_NG2 * _L + _L,), jnp.float32),
            pltpu.VMEM((2 * _L,), jnp.float32),
            pltpu.VMEM((3 * _L,), jnp.float32),
            pltpu.VMEM((_CAP + _L,), jnp.float32),
            pltpu.SemaphoreType.DMA,
            pltpu.SemaphoreType.DMA,
            pltpu.SemaphoreType.DMA,
        ],
    )(dists, taus)

    fused_body = functools.partial(_fused_body, nb=nb, tail=tail, bk=bk)
    out_tc = pl.pallas_call(
        fused_body,
        grid=(nb,),
        in_specs=[
            pl.BlockSpec((ntc, dim), lambda i: (0, 0)),
            pl.BlockSpec((bk, dim), lambda i: (i, 0)),
            pl.BlockSpec((1, _TW), lambda i: (0, 0)),
        ],
        out_specs=pl.BlockSpec((ntc, 1), lambda i: (0, 0)),
        out_shape=jax.ShapeDtypeStruct((ntc, 1), jnp.float32),
        scratch_shapes=[
            pltpu.VMEM((ntc, _TW), jnp.float32),
            pltpu.VMEM((ntc, 1), jnp.float32),
        ],
    )(q_tc, buffer_features, scale)

    out_sc = pl.pallas_call(
        _merge_body,
        in_specs=[
            pl.BlockSpec((nsc, _CAP), lambda: (0, 0)),
            pl.BlockSpec((nsc, _TW), lambda: (0, 0)),
            pl.BlockSpec((1, _TW), lambda: (0, 0)),
        ],
        out_specs=pl.BlockSpec((nsc, 1), lambda: (0, 0)),
        out_shape=jax.ShapeDtypeStruct((nsc, 1), jnp.float32),
    )(cand, taus, scale)
    return jnp.concatenate([out_sc[:, 0], out_tc[:, 0]])
